# baseline (speedup 1.0000x reference)
import functools

import jax
import jax.numpy as jnp
from jax import lax
from jax.experimental import pallas as pl
from jax.experimental.pallas import tpu as pltpu
from jax.experimental.pallas import tpu_sc as plsc

NN = 10000
NPAD = 10240
D = 128
E = 320000
BATCH = 4096
NC, NS = 2, 16
NW = NC * NS
CH = 128
CPT = 80
CPT0 = 80
CPT1 = 80
EPAD = NW * CPT * CH
RPT = NPAD // NS

_mesh = plsc.VectorSubcoreMesh(core_axis_name="c", subcore_axis_name="s")


@functools.partial(
    pl.kernel,
    out_type=[
        jax.ShapeDtypeStruct((NC, NPAD), jnp.float32),
        jax.ShapeDtypeStruct((BATCH, D), jnp.float32),
    ],
    mesh=_mesh,
    scratch_types=[
        pltpu.VMEM((CPT, CH), jnp.int32),
        pltpu.VMEM((NPAD,), jnp.float32),
        pltpu.VMEM((NS, RPT), jnp.float32),
        pltpu.VMEM((RPT,), jnp.float32),
        pltpu.VMEM((CH,), jnp.int32),
        pltpu.VMEM((CH, D), jnp.float32),
        pltpu.VMEM_SHARED((NS, NPAD), jnp.float32),
        pltpu.SemaphoreType.DMA,
    ],
    compiler_params=pltpu.CompilerParams(needs_layout_passes=False),
)
def _deg_users_kernel(dstp, u_idx, user_table, deg_out, users_out,
                      idxd, hist, cols, osum, idxu, urows, shared, sem):
    c = lax.axis_index("c")
    s = lax.axis_index("s")
    wid = c * NS + s

    pltpu.sync_copy(dstp.at[wid], idxd)

    pltpu.sync_copy(u_idx.at[wid], idxu)
    pltpu.async_copy(user_table.at[idxu], urows, sem).wait()
    pltpu.sync_copy(urows, users_out.at[pl.ds(wid * CH, CH)])

    z16 = jnp.zeros((16,), jnp.float32)

    def zr(j, _):
        hist[pl.ds(j * 16, 16)] = z16
        return 0

    lax.fori_loop(0, NPAD // 16, zr, 0)

    ones16 = jnp.full((16,), 1.0, jnp.float32)

    def hbody(j, _):
        for k in range(CH // 16):
            iv = idxd[j, pl.ds(k * 16, 16)]
            plsc.addupdate_scatter(hist, [iv], ones16)
        return 0

    lax.fori_loop(0, CPT, hbody, 0)

    pltpu.sync_copy(hist, shared.at[s])
    plsc.subcore_barrier()
    pltpu.sync_copy(shared.at[:, pl.ds(s * RPT, RPT)], cols)

    def rbody(j, _):
        v = cols[0, pl.ds(j * 16, 16)]
        for t in range(1, NS):
            v = v + cols[t, pl.ds(j * 16, 16)]
        osum[pl.ds(j * 16, 16)] = v
        return 0

    lax.fori_loop(0, RPT // 16, rbody, 0)
    pltpu.sync_copy(osum, deg_out.at[c, pl.ds(s * RPT, RPT)])


def _scatter_serial(srcp, dstp, z, acc, idxs, idxd, rows, sem, s, cpt):
    pltpu.sync_copy(srcp.at[s], idxs.at[pl.ds(0, cpt)])
    pltpu.sync_copy(dstp.at[s], idxd.at[pl.ds(0, cpt)])

    def body(j, _):
        pltpu.async_copy(z.at[idxs.at[j]], rows, sem).wait()
        pltpu.sync_copy(rows, acc.at[idxd.at[j]], add=True)
        return 0

    lax.fori_loop(0, cpt, body, 0)


@functools.partial(
    pl.kernel,
    out_type=jax.ShapeDtypeStruct((NC, NPAD, D), jnp.float32),
    mesh=_mesh,
    scratch_types=[
        pltpu.VMEM((CPT0, CH), jnp.int32),
        pltpu.VMEM((CPT0, CH), jnp.int32),
        pltpu.VMEM((CH, D), jnp.float32),
        pltpu.VMEM_SHARED((NPAD, D), jnp.float32),
        pltpu.SemaphoreType.DMA,
    ],
)
def _edge_scatter_kernel(src0, dst0, src1, dst1, z, zrows, out, idxs, idxd,
                         rows, acc, sem):
    c = lax.axis_index("c")
    s = lax.axis_index("s")

    pltpu.sync_copy(zrows, rows)
    for t in range(RPT // CH):
        pltpu.sync_copy(rows, acc.at[pl.ds(s * RPT + t * CH, CH)])
    plsc.subcore_barrier()

    @pl.when(c == 0)
    def _():
        _scatter_serial(src0, dst0, z, acc, idxs, idxd, rows, sem, s, CPT0)

    @pl.when(c == 1)
    def _():
        _scatter_serial(src1, dst1, z, acc, idxs, idxd, rows, sem, s, CPT1)

    plsc.subcore_barrier()

    pltpu.sync_copy(acc.at[pl.ds(s * RPT, RPT)],
                    out.at[c, pl.ds(s * RPT, RPT)])


@functools.partial(
    pl.kernel,
    out_type=jax.ShapeDtypeStruct((BATCH, D), jnp.float32),
    mesh=_mesh,
    scratch_types=[
        pltpu.VMEM((CH,), jnp.int32),
        pltpu.VMEM((CH, D), jnp.float32),
        pltpu.SemaphoreType.DMA,
    ],
)
def _gather_kernel(idx_hbm, table, out, idxv, rows, sem):
    wid = lax.axis_index("c") * NS + lax.axis_index("s")
    pltpu.sync_copy(idx_hbm.at[wid], idxv)
    pltpu.async_copy(table.at[idxv], rows, sem).wait()
    pltpu.sync_copy(rows, out.at[pl.ds(wid * CH, CH)])


def _dis(deg_ref):
    d = deg_ref[0] + deg_ref[1] + 1.0
    return lax.rsqrt(d)


def _renorm_block(x):
    n = jnp.sqrt(jnp.sum(x * x, axis=1, keepdims=True))
    return x * jnp.where(n > 1.0, 1.0 / (n + 1e-7), 1.0)


def _mm(a, b):
    return lax.dot_general(a, b, (((1,), (0,)), ((), ())),
                           preferred_element_type=jnp.float32,
                           precision=lax.Precision.HIGHEST)


_BR = 2048
_GRID = NPAD // _BR


def _k1_body(ent_ref, deg_ref, w_ref, out_ref):
    x = _renorm_block(ent_ref[...])
    out_ref[...] = _mm(x, w_ref[...]) * _dis(deg_ref)


def _k3_body(z_ref, s_ref, deg_ref, b_ref, w_ref, out_ref):
    dis = _dis(deg_ref)
    h = (z_ref[...] + s_ref[0] + s_ref[1]) * dis + b_ref[...]
    out_ref[...] = _mm(jnp.maximum(h, 0.0), w_ref[...]) * dis


def _k5_body(z_ref, s_ref, deg_ref, b_ref, out_ref):
    out_ref[...] = (z_ref[...] + s_ref[0] + s_ref[1]) * _dis(deg_ref) + b_ref[...]


def _k7_body(u_ref, it_ref, out_ref):
    us = _renorm_block(u_ref[...])
    uv = jnp.sum(us * it_ref[...], axis=1, keepdims=True)
    out_ref[...] = jax.nn.sigmoid(uv)


def _row_spec(r3=False):
    if r3:
        return pl.BlockSpec((NC, _BR, D), lambda r: (0, r, 0))
    return pl.BlockSpec((_BR, D), lambda r: (r, 0))


_DEG_SPEC = pl.BlockSpec((NC, _BR, 1), lambda r: (0, r, 0))
_W_SPEC = pl.BlockSpec((D, D), lambda r: (0, 0))
_B_SPEC = pl.BlockSpec((1, D), lambda r: (0, 0))


def kernel(u, i, edge_index, user_table, entity_table, W1, b1, W2, b2):
    src = edge_index[0].astype(jnp.int32)
    dst = edge_index[1].astype(jnp.int32)
    pad = jnp.full((EPAD - E,), NN, jnp.int32)
    srcf = jnp.concatenate([src, pad])
    dstf = jnp.concatenate([dst, pad])
    dstp = dstf.reshape(NW, CPT, CH)
    n0 = NS * CPT0 * CH
    src0 = srcf[:n0].reshape(NS, CPT0, CH)
    dst0 = dstf[:n0].reshape(NS, CPT0, CH)
    src1 = srcf[n0:].reshape(NS, CPT1, CH)
    dst1 = dstf[n0:].reshape(NS, CPT1, CH)
    ent = jnp.concatenate(
        [entity_table, jnp.zeros((NPAD - NN, D), jnp.float32)], axis=0)
    u2 = u.astype(jnp.int32).reshape(NW, CH)
    i2 = i.astype(jnp.int32).reshape(NW, CH)
    b1r = b1.reshape(1, D)
    b2r = b2.reshape(1, D)
    zrows = jnp.zeros((CH, D), jnp.float32)

    deg2, users_raw = _deg_users_kernel(dstp, u2, user_table)
    deg2 = deg2.reshape(NC, NPAD, 1)

    z1 = pl.pallas_call(
        _k1_body, grid=(_GRID,),
        in_specs=[_row_spec(), _DEG_SPEC, _W_SPEC],
        out_specs=_row_spec(),
        out_shape=jax.ShapeDtypeStruct((NPAD, D), jnp.float32),
    )(ent, deg2, W1)

    s1 = _edge_scatter_kernel(src0, dst0, src1, dst1, z1, zrows)

    z2 = pl.pallas_call(
        _k3_body, grid=(_GRID,),
        in_specs=[_row_spec(), _row_spec(True), _DEG_SPEC, _B_SPEC, _W_SPEC],
        out_specs=_row_spec(),
        out_shape=jax.ShapeDtypeStruct((NPAD, D), jnp.float32),
    )(z1, s1, deg2, b1r, W2)

    s2 = _edge_scatter_kernel(src0, dst0, src1, dst1, z2, zrows)

    x2 = pl.pallas_call(
        _k5_body, grid=(_GRID,),
        in_specs=[_row_spec(), _row_spec(True), _DEG_SPEC, _B_SPEC],
        out_specs=_row_spec(),
        out_shape=jax.ShapeDtypeStruct((NPAD, D), jnp.float32),
    )(z2, s2, deg2, b2r)

    items = _gather_kernel(i2, x2)

    uv = pl.pallas_call(
        _k7_body, grid=(2,),
        in_specs=[pl.BlockSpec((BATCH // 2, D), lambda r: (r, 0)),
                  pl.BlockSpec((BATCH // 2, D), lambda r: (r, 0))],
        out_specs=pl.BlockSpec((BATCH // 2, 1), lambda r: (r, 0)),
        out_shape=jax.ShapeDtypeStruct((BATCH, 1), jnp.float32),
    )(users_raw, items)

    return uv.reshape(BATCH)


# speedup vs baseline: 1.1731x; 1.1731x over previous
import functools

import jax
import jax.numpy as jnp
from jax import lax
from jax.experimental import pallas as pl
from jax.experimental.pallas import tpu as pltpu
from jax.experimental.pallas import tpu_sc as plsc

NN = 10000
NPAD = 10240
D = 128
E = 320000
BATCH = 4096
NC, NS = 2, 16
NW = NC * NS
CH = 128
CPT = 80
EPAD = NW * CPT * CH
RPT = NPAD // NS

_mesh = plsc.VectorSubcoreMesh(core_axis_name="c", subcore_axis_name="s")


@functools.partial(
    pl.kernel,
    out_type=[
        jax.ShapeDtypeStruct((NC, NPAD), jnp.float32),
        jax.ShapeDtypeStruct((BATCH, D), jnp.float32),
    ],
    mesh=_mesh,
    scratch_types=[
        pltpu.VMEM((CPT, CH), jnp.int32),
        pltpu.VMEM((NPAD,), jnp.float32),
        pltpu.VMEM((NS, RPT), jnp.float32),
        pltpu.VMEM((RPT,), jnp.float32),
        pltpu.VMEM((CH,), jnp.int32),
        pltpu.VMEM((CH, D), jnp.float32),
        pltpu.VMEM_SHARED((NS, NPAD), jnp.float32),
        pltpu.SemaphoreType.DMA,
    ],
    compiler_params=pltpu.CompilerParams(needs_layout_passes=False),
)
def _deg_users_kernel(dstp, u_idx, user_table, deg_out, users_out,
                      idxd, hist, cols, osum, idxu, urows, shared, sem):
    c = lax.axis_index("c")
    s = lax.axis_index("s")
    wid = c * NS + s

    pltpu.sync_copy(dstp.at[wid], idxd)

    pltpu.sync_copy(u_idx.at[wid], idxu)
    pltpu.async_copy(user_table.at[idxu], urows, sem).wait()
    pltpu.sync_copy(urows, users_out.at[pl.ds(wid * CH, CH)])

    z16 = jnp.zeros((16,), jnp.float32)

    def zr(j, _):
        hist[pl.ds(j * 16, 16)] = z16
        return 0

    lax.fori_loop(0, NPAD // 16, zr, 0)

    ones16 = jnp.full((16,), 1.0, jnp.float32)

    def hbody(j, _):
        for k in range(CH // 16):
            iv = idxd[j, pl.ds(k * 16, 16)]
            plsc.addupdate_scatter(hist, [iv], ones16)
        return 0

    lax.fori_loop(0, CPT, hbody, 0)

    pltpu.sync_copy(hist, shared.at[s])
    plsc.subcore_barrier()
    pltpu.sync_copy(shared.at[:, pl.ds(s * RPT, RPT)], cols)

    def rbody(j, _):
        v = cols[0, pl.ds(j * 16, 16)]
        for t in range(1, NS):
            v = v + cols[t, pl.ds(j * 16, 16)]
        osum[pl.ds(j * 16, 16)] = v
        return 0

    lax.fori_loop(0, RPT // 16, rbody, 0)
    pltpu.sync_copy(osum, deg_out.at[c, pl.ds(s * RPT, RPT)])


@functools.partial(
    pl.kernel,
    out_type=jax.ShapeDtypeStruct((NC, NPAD, D), jnp.float32),
    mesh=_mesh,
    scratch_types=[
        pltpu.VMEM((CPT, CH), jnp.int32),
        pltpu.VMEM((CPT, CH), jnp.int32),
        pltpu.VMEM((CH, D), jnp.float32),
        pltpu.VMEM_SHARED((NPAD, D), jnp.float32),
        pltpu.SemaphoreType.DMA,
    ],
)
def _edge_scatter_kernel(srcp, dstp, z, zrows, out, idxs, idxd,
                         rows, acc, sem):
    c = lax.axis_index("c")
    s = lax.axis_index("s")
    wid = c * NS + s

    pltpu.sync_copy(zrows, rows)
    for t in range(RPT // CH):
        pltpu.sync_copy(rows, acc.at[pl.ds(s * RPT + t * CH, CH)])
    plsc.subcore_barrier()

    pltpu.sync_copy(srcp.at[wid], idxs)
    pltpu.sync_copy(dstp.at[wid], idxd)

    def body(j, _):
        pltpu.async_copy(z.at[idxs.at[j]], rows, sem).wait()
        pltpu.sync_copy(rows, acc.at[idxd.at[j]], add=True)
        return 0

    lax.fori_loop(0, CPT, body, 0)
    plsc.subcore_barrier()

    pltpu.sync_copy(acc.at[pl.ds(s * RPT, RPT)],
                    out.at[c, pl.ds(s * RPT, RPT)])


@functools.partial(
    pl.kernel,
    out_type=jax.ShapeDtypeStruct((BATCH, D), jnp.float32),
    mesh=_mesh,
    scratch_types=[
        pltpu.VMEM((CH,), jnp.int32),
        pltpu.VMEM((CH, D), jnp.float32),
        pltpu.SemaphoreType.DMA,
    ],
)
def _gather_kernel(idx_hbm, table, out, idxv, rows, sem):
    wid = lax.axis_index("c") * NS + lax.axis_index("s")
    pltpu.sync_copy(idx_hbm.at[wid], idxv)
    pltpu.async_copy(table.at[idxv], rows, sem).wait()
    pltpu.sync_copy(rows, out.at[pl.ds(wid * CH, CH)])


def _dis(deg_ref):
    d = deg_ref[0] + deg_ref[1] + 1.0
    return lax.rsqrt(d)


def _renorm_block(x):
    n = jnp.sqrt(jnp.sum(x * x, axis=1, keepdims=True))
    return x * jnp.where(n > 1.0, 1.0 / (n + 1e-7), 1.0)


def _mm(a, b):
    return lax.dot_general(a, b, (((1,), (0,)), ((), ())),
                           preferred_element_type=jnp.float32,
                           precision=lax.Precision.HIGHEST)


_BR = 2048
_GRID = NPAD // _BR


def _k1_body(ent_ref, deg_ref, w_ref, out_ref):
    x = _renorm_block(ent_ref[...])
    out_ref[...] = _mm(x, w_ref[...]) * _dis(deg_ref)


def _k3_body(z_ref, s_ref, deg_ref, b_ref, w_ref, out_ref):
    dis = _dis(deg_ref)
    h = (z_ref[...] + s_ref[0] + s_ref[1]) * dis + b_ref[...]
    out_ref[...] = _mm(jnp.maximum(h, 0.0), w_ref[...]) * dis


def _k5_body(z_ref, s_ref, deg_ref, b_ref, out_ref):
    out_ref[...] = (z_ref[...] + s_ref[0] + s_ref[1]) * _dis(deg_ref) + b_ref[...]


def _k7_body(u_ref, it_ref, out_ref):
    us = _renorm_block(u_ref[...])
    uv = jnp.sum(us * it_ref[...], axis=1, keepdims=True)
    out_ref[...] = jax.nn.sigmoid(uv)


def _row_spec(r3=False):
    if r3:
        return pl.BlockSpec((NC, _BR, D), lambda r: (0, r, 0))
    return pl.BlockSpec((_BR, D), lambda r: (r, 0))


_DEG_SPEC = pl.BlockSpec((NC, _BR, 1), lambda r: (0, r, 0))
_W_SPEC = pl.BlockSpec((D, D), lambda r: (0, 0))
_B_SPEC = pl.BlockSpec((1, D), lambda r: (0, 0))


def kernel(u, i, edge_index, user_table, entity_table, W1, b1, W2, b2):
    src = edge_index[0].astype(jnp.int32)
    dst = edge_index[1].astype(jnp.int32)
    pad = jnp.full((EPAD - E,), NN, jnp.int32)
    srcf = jnp.concatenate([src, pad])
    dstf = jnp.concatenate([dst, pad])
    dstp = dstf.reshape(NW, CPT, CH)
    srcp = srcf.reshape(NW, CPT, CH)
    ent = jnp.concatenate(
        [entity_table, jnp.zeros((NPAD - NN, D), jnp.float32)], axis=0)
    u2 = u.astype(jnp.int32).reshape(NW, CH)
    i2 = i.astype(jnp.int32).reshape(NW, CH)
    b1r = b1.reshape(1, D)
    b2r = b2.reshape(1, D)
    zrows = jnp.zeros((CH, D), jnp.float32)

    deg2, users_raw = _deg_users_kernel(dstp, u2, user_table)
    deg2 = deg2.reshape(NC, NPAD, 1)

    z1 = pl.pallas_call(
        _k1_body, grid=(_GRID,),
        in_specs=[_row_spec(), _DEG_SPEC, _W_SPEC],
        out_specs=_row_spec(),
        out_shape=jax.ShapeDtypeStruct((NPAD, D), jnp.float32),
    )(ent, deg2, W1)

    s1 = _edge_scatter_kernel(srcp, dstp, z1, zrows)

    z2 = pl.pallas_call(
        _k3_body, grid=(_GRID,),
        in_specs=[_row_spec(), _row_spec(True), _DEG_SPEC, _B_SPEC, _W_SPEC],
        out_specs=_row_spec(),
        out_shape=jax.ShapeDtypeStruct((NPAD, D), jnp.float32),
    )(z1, s1, deg2, b1r, W2)

    s2 = _edge_scatter_kernel(srcp, dstp, z2, zrows)

    x2 = pl.pallas_call(
        _k5_body, grid=(_GRID,),
        in_specs=[_row_spec(), _row_spec(True), _DEG_SPEC, _B_SPEC],
        out_specs=_row_spec(),
        out_shape=jax.ShapeDtypeStruct((NPAD, D), jnp.float32),
    )(z2, s2, deg2, b2r)

    items = _gather_kernel(i2, x2)

    uv = pl.pallas_call(
        _k7_body, grid=(2,),
        in_specs=[pl.BlockSpec((BATCH // 2, D), lambda r: (r, 0)),
                  pl.BlockSpec((BATCH // 2, D), lambda r: (r, 0))],
        out_specs=pl.BlockSpec((BATCH // 2, 1), lambda r: (r, 0)),
        out_shape=jax.ShapeDtypeStruct((BATCH, 1), jnp.float32),
    )(users_raw, items)

    return uv.reshape(BATCH)


# speedup vs baseline: 1.7284x; 1.4734x over previous
import functools

import jax
import jax.numpy as jnp
from jax import lax
from jax.experimental import pallas as pl
from jax.experimental.pallas import tpu as pltpu
from jax.experimental.pallas import tpu_sc as plsc

NN = 10000
NPAD = 10240
D = 128
E = 320000
BATCH = 4096
NC, NS = 2, 16
NW = NC * NS
CH = 128
CPT = 79
EPAD = NW * CPT * CH
RPT = NPAD // NS

_mesh = plsc.VectorSubcoreMesh(core_axis_name="c", subcore_axis_name="s")


@functools.partial(
    pl.kernel,
    out_type=[
        jax.ShapeDtypeStruct((NC, NPAD), jnp.float32),
        jax.ShapeDtypeStruct((BATCH, D), jnp.float32),
    ],
    mesh=_mesh,
    scratch_types=[
        pltpu.VMEM((CPT, CH), jnp.int32),
        pltpu.VMEM((NPAD,), jnp.float32),
        pltpu.VMEM((NS, RPT), jnp.float32),
        pltpu.VMEM((RPT,), jnp.float32),
        pltpu.VMEM((CH,), jnp.int32),
        pltpu.VMEM((CH, D), jnp.float32),
        pltpu.VMEM_SHARED((NS, NPAD), jnp.float32),
        pltpu.SemaphoreType.DMA,
    ],
    compiler_params=pltpu.CompilerParams(needs_layout_passes=False),
)
def _deg_users_kernel(dstp, u_idx, user_table, deg_out, users_out,
                      idxd, hist, cols, osum, idxu, urows, shared, sem):
    c = lax.axis_index("c")
    s = lax.axis_index("s")
    wid = c * NS + s

    pltpu.sync_copy(dstp.at[wid], idxd)

    pltpu.sync_copy(u_idx.at[wid], idxu)
    pltpu.async_copy(user_table.at[idxu], urows, sem).wait()
    pltpu.sync_copy(urows, users_out.at[pl.ds(wid * CH, CH)])

    z16 = jnp.zeros((16,), jnp.float32)

    def zr(j, _):
        hist[pl.ds(j * 16, 16)] = z16
        return 0

    lax.fori_loop(0, NPAD // 16, zr, 0)

    ones16 = jnp.full((16,), 1.0, jnp.float32)

    def hbody(j, _):
        for k in range(CH // 16):
            iv = idxd[j, pl.ds(k * 16, 16)]
            plsc.addupdate_scatter(hist, [iv], ones16)
        return 0

    lax.fori_loop(0, CPT, hbody, 0)

    pltpu.sync_copy(hist, shared.at[s])
    plsc.subcore_barrier()
    pltpu.sync_copy(shared.at[:, pl.ds(s * RPT, RPT)], cols)

    def rbody(j, _):
        v = cols[0, pl.ds(j * 16, 16)]
        for t in range(1, NS):
            v = v + cols[t, pl.ds(j * 16, 16)]
        osum[pl.ds(j * 16, 16)] = v
        return 0

    lax.fori_loop(0, RPT // 16, rbody, 0)
    pltpu.sync_copy(osum, deg_out.at[c, pl.ds(s * RPT, RPT)])


@functools.partial(
    pl.kernel,
    out_type=jax.ShapeDtypeStruct((NC, NPAD, D), jnp.float32),
    mesh=_mesh,
    scratch_types=[
        pltpu.VMEM((CPT, CH), jnp.int32),
        pltpu.VMEM((CPT, CH), jnp.int32),
        pltpu.VMEM((CH, D), jnp.float32),
        pltpu.VMEM_SHARED((NPAD, D), jnp.float32),
        pltpu.SemaphoreType.DMA,
    ],
)
def _edge_scatter_kernel(srcp, dstp, z, zrows, out, idxs, idxd,
                         rows, acc, sem):
    c = lax.axis_index("c")
    s = lax.axis_index("s")
    wid = c * NS + s

    z16 = jnp.zeros((16,), jnp.float32)

    def zrow(j, _):
        for k in range(D // 16):
            rows[j, pl.ds(k * 16, 16)] = z16
        return 0

    lax.fori_loop(0, CH, zrow, 0)
    for t in range(RPT // CH):
        pltpu.sync_copy(rows, acc.at[pl.ds(s * RPT + t * CH, CH)])
    plsc.subcore_barrier()

    pltpu.sync_copy(srcp.at[wid], idxs)
    pltpu.sync_copy(dstp.at[wid], idxd)

    def body(j, _):
        pltpu.async_copy(z.at[idxs.at[j]], rows, sem).wait()
        pltpu.sync_copy(rows, acc.at[idxd.at[j]], add=True)
        return 0

    lax.fori_loop(0, CPT, body, 0)
    plsc.subcore_barrier()

    pltpu.sync_copy(acc.at[pl.ds(s * RPT, RPT)],
                    out.at[c, pl.ds(s * RPT, RPT)])


@functools.partial(
    pl.kernel,
    out_type=jax.ShapeDtypeStruct((BATCH, D), jnp.float32),
    mesh=_mesh,
    scratch_types=[
        pltpu.VMEM((CH,), jnp.int32),
        pltpu.VMEM((CH, D), jnp.float32),
        pltpu.SemaphoreType.DMA,
    ],
)
def _gather_kernel(idx_hbm, table, out, idxv, rows, sem):
    wid = lax.axis_index("c") * NS + lax.axis_index("s")
    pltpu.sync_copy(idx_hbm.at[wid], idxv)
    pltpu.async_copy(table.at[idxv], rows, sem).wait()
    pltpu.sync_copy(rows, out.at[pl.ds(wid * CH, CH)])


def _dis(deg_ref):
    d = deg_ref[0] + deg_ref[1] + 1.0
    return lax.rsqrt(d)


def _renorm_block(x):
    n = jnp.sqrt(jnp.sum(x * x, axis=1, keepdims=True))
    return x * jnp.where(n > 1.0, 1.0 / (n + 1e-7), 1.0)


def _mm(a, b):
    return lax.dot_general(a, b, (((1,), (0,)), ((), ())),
                           preferred_element_type=jnp.float32,
                           precision=lax.Precision.HIGHEST)


_BR = 2048
_GRID = NPAD // _BR


def _k1_body(ent_ref, deg_ref, w_ref, out_ref):
    x = _renorm_block(ent_ref[...])
    out_ref[...] = _mm(x, w_ref[...]) * _dis(deg_ref)


def _k3_body(z_ref, s_ref, deg_ref, b_ref, w_ref, out_ref):
    dis = _dis(deg_ref)
    h = (z_ref[...] + s_ref[0] + s_ref[1]) * dis + b_ref[...]
    out_ref[...] = _mm(jnp.maximum(h, 0.0), w_ref[...]) * dis


def _k5_body(z_ref, s_ref, deg_ref, b_ref, out_ref):
    out_ref[...] = (z_ref[...] + s_ref[0] + s_ref[1]) * _dis(deg_ref) + b_ref[...]


def _k7_body(u_ref, it_ref, out_ref):
    us = _renorm_block(u_ref[...])
    uv = jnp.sum(us * it_ref[...], axis=1, keepdims=True)
    out_ref[...] = jax.nn.sigmoid(uv)


def _row_spec(r3=False):
    if r3:
        return pl.BlockSpec((NC, _BR, D), lambda r: (0, r, 0))
    return pl.BlockSpec((_BR, D), lambda r: (r, 0))


_DEG_SPEC = pl.BlockSpec((NC, _BR, 1), lambda r: (0, r, 0))
_W_SPEC = pl.BlockSpec((D, D), lambda r: (0, 0))
_B_SPEC = pl.BlockSpec((1, D), lambda r: (0, 0))


def kernel(u, i, edge_index, user_table, entity_table, W1, b1, W2, b2):
    src = edge_index[0].astype(jnp.int32)
    dst = edge_index[1].astype(jnp.int32)
    pad = jnp.full((EPAD - E,), NN, jnp.int32)
    srcf = jnp.concatenate([src, pad])
    dstf = jnp.concatenate([dst, pad])
    dstp = dstf.reshape(NW, CPT, CH)
    srcp = srcf.reshape(NW, CPT, CH)
    ent = jnp.concatenate(
        [entity_table, jnp.zeros((NPAD - NN, D), jnp.float32)], axis=0)
    u2 = u.astype(jnp.int32).reshape(NW, CH)
    i2 = i.astype(jnp.int32).reshape(NW, CH)
    b1r = b1.reshape(1, D)
    b2r = b2.reshape(1, D)
    zrows = jnp.zeros((CH, D), jnp.float32)

    deg2, users_raw = _deg_users_kernel(dstp, u2, user_table)
    deg2 = deg2.reshape(NC, NPAD, 1)

    z1 = pl.pallas_call(
        _k1_body, grid=(_GRID,),
        in_specs=[_row_spec(), _DEG_SPEC, _W_SPEC],
        out_specs=_row_spec(),
        out_shape=jax.ShapeDtypeStruct((NPAD, D), jnp.float32),
    )(ent, deg2, W1)

    s1 = _edge_scatter_kernel(srcp, dstp, z1, zrows)

    z2 = pl.pallas_call(
        _k3_body, grid=(_GRID,),
        in_specs=[_row_spec(), _row_spec(True), _DEG_SPEC, _B_SPEC, _W_SPEC],
        out_specs=_row_spec(),
        out_shape=jax.ShapeDtypeStruct((NPAD, D), jnp.float32),
    )(z1, s1, deg2, b1r, W2)

    s2 = _edge_scatter_kernel(srcp, dstp, z2, zrows)

    x2 = pl.pallas_call(
        _k5_body, grid=(_GRID,),
        in_specs=[_row_spec(), _row_spec(True), _DEG_SPEC, _B_SPEC],
        out_specs=_row_spec(),
        out_shape=jax.ShapeDtypeStruct((NPAD, D), jnp.float32),
    )(z2, s2, deg2, b2r)

    items = _gather_kernel(i2, x2)

    uv = pl.pallas_call(
        _k7_body, grid=(2,),
        in_specs=[pl.BlockSpec((BATCH // 2, D), lambda r: (r, 0)),
                  pl.BlockSpec((BATCH // 2, D), lambda r: (r, 0))],
        out_specs=pl.BlockSpec((BATCH // 2, 1), lambda r: (r, 0)),
        out_shape=jax.ShapeDtypeStruct((BATCH, 1), jnp.float32),
    )(users_raw, items)

    return uv.reshape(BATCH)
